# baseline (speedup 1.0000x reference)
import functools

import jax
import jax.numpy as jnp
from jax.experimental import pallas as pl
from jax.experimental.pallas import tpu as pltpu
from jax.experimental.pallas import tpu_sc as plsc

_QUART = 1 << 18
_COL_BLOCK = 8192
_GATHER_WINDOW = 128
_BATCH_BLOCK = 1024
_CHUNKS = 4


def _pack_bf16(a_ref, b_ref):
    au = jax.lax.bitcast_convert_type(a_ref[...], jnp.uint32)
    bu = jax.lax.bitcast_convert_type(b_ref[...], jnp.uint32)
    return (au & jnp.uint32(0xFFFF0000)) | (bu >> 16)


def _relayout_kernel(p0_ref, p1_ref, p2_ref, p3_ref, p_ref):
    stacked = jnp.concatenate(
        [_pack_bf16(p0_ref, p1_ref), _pack_bf16(p2_ref, p3_ref)], axis=0
    )
    packed = jnp.swapaxes(stacked, 0, 1)
    p_ref[...] = jax.lax.bitcast_convert_type(packed, jnp.int32)


def _tc_make_table(Et):
    emb, vocab = Et.shape
    n_q = _QUART // _COL_BLOCK
    n_all = pl.cdiv(vocab, _COL_BLOCK)

    def part_map(p):
        def index_map(i):
            return (0, jnp.minimum(i + p * n_q, n_all - 1))
        return index_map

    return pl.pallas_call(
        _relayout_kernel,
        grid=(n_q,),
        in_specs=[
            pl.BlockSpec((emb, _COL_BLOCK), part_map(p)) for p in range(4)
        ],
        out_specs=pl.BlockSpec((_COL_BLOCK, 2 * emb), lambda i: (i, 0)),
        out_shape=jax.ShapeDtypeStruct((_QUART, 2 * emb), jnp.int32),
        compiler_params=pltpu.CompilerParams(
            dimension_semantics=("parallel",),
        ),
    )(Et, Et, Et, Et)


def _sc_gather(P, idx):
    n = idx.shape[1]
    width = P.shape[1]
    mesh = plsc.VectorSubcoreMesh(core_axis_name="core", subcore_axis_name="subcore")

    @functools.partial(
        pl.kernel,
        out_type=jax.ShapeDtypeStruct((n, width), P.dtype),
        mesh=mesh,
    )
    def gather_kernel(table_hbm, idx_hbm, out_hbm):
        def body(idx_vmem, out_vmem):
            pltpu.sync_copy(table_hbm.at[idx_vmem.at[0]], out_vmem)

        pltpu.emit_pipeline(
            body,
            grid=(n // _GATHER_WINDOW,),
            in_specs=[
                pl.BlockSpec((1, _GATHER_WINDOW), index_map=lambda i: (0, i))
            ],
            out_specs=[
                pl.BlockSpec((_GATHER_WINDOW, width), index_map=lambda i: (i, 0))
            ],
            core_axis_name=("core", "subcore"),
            dimension_semantics=(pltpu.PARALLEL,),
        )(idx_hbm, out_hbm)

    return gather_kernel(P, idx)


def _mlp_kernel(g_ref, xt_ref, w1_ref, b1_ref, w2_ref, b2_ref, o_ref):
    k_dim, bb, width = g_ref.shape
    emb = width // 2
    lane = jax.lax.broadcasted_iota(jnp.int32, (bb, width), 1)
    parts = []
    for k in range(k_dim):
        gk = jax.lax.bitcast_convert_type(g_ref[k], jnp.uint32)
        part = xt_ref[k][:, None] >> 18
        keep = (lane >= emb) == (part >= 2)
        shifted = (gk << ((part & 1).astype(jnp.uint32) * 16)) \
            & jnp.uint32(0xFFFF0000)
        val = jax.lax.bitcast_convert_type(
            jnp.where(keep, shifted, jnp.uint32(0)), jnp.float32)
        parts.append(val.astype(jnp.bfloat16))
    f = jnp.concatenate(parts, axis=1)
    h = jnp.dot(f, w1_ref[...], preferred_element_type=jnp.float32)
    h = jnp.maximum(h + b1_ref[...], 0.0)
    o = jnp.dot(h, w2_ref[...], preferred_element_type=jnp.float32)
    o_ref[...] = o + b2_ref[...]


def _tc_mlp(g, xt, W1d, b1, W2, b2, start, rows):
    k_dim, b, width = g.shape
    hid = W1d.shape[1]
    out = W2.shape[1]
    s0 = start // _BATCH_BLOCK
    return pl.pallas_call(
        _mlp_kernel,
        grid=(rows // _BATCH_BLOCK,),
        in_specs=[
            pl.BlockSpec((k_dim, _BATCH_BLOCK, width), lambda i: (0, i + s0, 0)),
            pl.BlockSpec((k_dim, _BATCH_BLOCK), lambda i: (0, i + s0)),
            pl.BlockSpec((k_dim * width, hid), lambda i: (0, 0)),
            pl.BlockSpec((1, hid), lambda i: (0, 0)),
            pl.BlockSpec((hid, out), lambda i: (0, 0)),
            pl.BlockSpec((1, out), lambda i: (0, 0)),
        ],
        out_specs=pl.BlockSpec((_BATCH_BLOCK, out), lambda i: (i, 0)),
        out_shape=jax.ShapeDtypeStruct((rows, out), jnp.float32),
        compiler_params=pltpu.CompilerParams(
            dimension_semantics=("parallel",),
        ),
    )(g, xt, W1d, b1, W2, b2)


def kernel(x, E, W1, b1, W2, b2):
    batch, k = x.shape
    vocab, emb = E.shape
    hid = W1.shape[1]

    P = _tc_make_table(E.T)

    xt = x.T.astype(jnp.int32)

    W1k = W1.reshape(k, emb, hid)
    W1d = jnp.concatenate([W1k, W1k], axis=1)
    W1d = W1d.reshape(k * 2 * emb, hid).astype(jnp.bfloat16)

    sizes = (6144, 6144, 4096)
    offs = [0]
    for sz in sizes:
        offs.append(offs[-1] + sz)
    outs = []
    for c, bc in enumerate(sizes):
        xt_c = xt[:, offs[c]:offs[c] + bc]
        idx_c = (xt_c & (_QUART - 1)).reshape(1, bc * k)
        g = _sc_gather(P, idx_c)
        g = g.reshape(k, bc, 2 * emb)
        outs.append(_tc_mlp(
            g, xt_c, W1d, b1.reshape(1, -1), W2, b2.reshape(1, -1), 0, bc,
        ))
    return jnp.concatenate(outs, axis=0)


# speedup vs baseline: 1.0381x; 1.0381x over previous
import functools

import jax
import jax.numpy as jnp
from jax.experimental import pallas as pl
from jax.experimental.pallas import tpu as pltpu
from jax.experimental.pallas import tpu_sc as plsc

_QUART = 1 << 18
_COL_BLOCK = 4096
_GATHER_WINDOW = 256
_BATCH_BLOCK = 1024
_CHUNKS = 4


def _pack_bf16(a_ref, b_ref):
    au = jax.lax.bitcast_convert_type(a_ref[...], jnp.uint32)
    bu = jax.lax.bitcast_convert_type(b_ref[...], jnp.uint32)
    return (au & jnp.uint32(0xFFFF0000)) | (bu >> 16)


def _relayout_kernel(p0_ref, p1_ref, p2_ref, p3_ref, p_ref):
    stacked = jnp.concatenate(
        [_pack_bf16(p0_ref, p1_ref), _pack_bf16(p2_ref, p3_ref)], axis=0
    )
    packed = jnp.swapaxes(stacked, 0, 1)
    p_ref[...] = jax.lax.bitcast_convert_type(packed, jnp.int32)


def _tc_make_table(Et):
    emb, vocab = Et.shape
    n_q = _QUART // _COL_BLOCK
    n_all = pl.cdiv(vocab, _COL_BLOCK)

    def part_map(p):
        def index_map(i):
            return (0, jnp.minimum(i + p * n_q, n_all - 1))
        return index_map

    return pl.pallas_call(
        _relayout_kernel,
        grid=(n_q,),
        in_specs=[
            pl.BlockSpec((emb, _COL_BLOCK), part_map(p)) for p in range(4)
        ],
        out_specs=pl.BlockSpec((_COL_BLOCK, 2 * emb), lambda i: (i, 0)),
        out_shape=jax.ShapeDtypeStruct((_QUART, 2 * emb), jnp.int32),
        compiler_params=pltpu.CompilerParams(
            dimension_semantics=("parallel",),
        ),
    )(Et, Et, Et, Et)


def _sc_gather(P, idx):
    n = idx.shape[1]
    width = P.shape[1]
    mesh = plsc.VectorSubcoreMesh(core_axis_name="core", subcore_axis_name="subcore")

    @functools.partial(
        pl.kernel,
        out_type=jax.ShapeDtypeStruct((n, width), P.dtype),
        mesh=mesh,
    )
    def gather_kernel(table_hbm, idx_hbm, out_hbm):
        def body(idx_vmem, out_vmem):
            pltpu.sync_copy(table_hbm.at[idx_vmem.at[0]], out_vmem)

        pltpu.emit_pipeline(
            body,
            grid=(n // _GATHER_WINDOW,),
            in_specs=[
                pl.BlockSpec((1, _GATHER_WINDOW), index_map=lambda i: (0, i))
            ],
            out_specs=[
                pl.BlockSpec((_GATHER_WINDOW, width), index_map=lambda i: (i, 0))
            ],
            core_axis_name=("core", "subcore"),
            dimension_semantics=(pltpu.PARALLEL,),
        )(idx_hbm, out_hbm)

    return gather_kernel(P, idx)


def _mlp_kernel(g_ref, xt_ref, w1_ref, b1_ref, w2_ref, b2_ref, o_ref):
    k_dim, bb, width = g_ref.shape
    emb = width // 2
    lane = jax.lax.broadcasted_iota(jnp.int32, (bb, width), 1)
    parts = []
    for k in range(k_dim):
        gk = jax.lax.bitcast_convert_type(g_ref[k], jnp.uint32)
        part = xt_ref[k][:, None] >> 18
        keep = (lane >= emb) == (part >= 2)
        shifted = (gk << ((part & 1).astype(jnp.uint32) * 16)) \
            & jnp.uint32(0xFFFF0000)
        val = jax.lax.bitcast_convert_type(
            jnp.where(keep, shifted, jnp.uint32(0)), jnp.float32)
        parts.append(val.astype(jnp.bfloat16))
    f = jnp.concatenate(parts, axis=1)
    h = jnp.dot(f, w1_ref[...], preferred_element_type=jnp.float32)
    h = jnp.maximum(h + b1_ref[...], 0.0)
    o = jnp.dot(h, w2_ref[...], preferred_element_type=jnp.float32)
    o_ref[...] = o + b2_ref[...]


def _tc_mlp(g, xt, W1d, b1, W2, b2, start, rows):
    k_dim, b, width = g.shape
    hid = W1d.shape[1]
    out = W2.shape[1]
    s0 = start // _BATCH_BLOCK
    return pl.pallas_call(
        _mlp_kernel,
        grid=(rows // _BATCH_BLOCK,),
        in_specs=[
            pl.BlockSpec((k_dim, _BATCH_BLOCK, width), lambda i: (0, i + s0, 0)),
            pl.BlockSpec((k_dim, _BATCH_BLOCK), lambda i: (0, i + s0)),
            pl.BlockSpec((k_dim * width, hid), lambda i: (0, 0)),
            pl.BlockSpec((1, hid), lambda i: (0, 0)),
            pl.BlockSpec((hid, out), lambda i: (0, 0)),
            pl.BlockSpec((1, out), lambda i: (0, 0)),
        ],
        out_specs=pl.BlockSpec((_BATCH_BLOCK, out), lambda i: (i, 0)),
        out_shape=jax.ShapeDtypeStruct((rows, out), jnp.float32),
        compiler_params=pltpu.CompilerParams(
            dimension_semantics=("parallel",),
        ),
    )(g, xt, W1d, b1, W2, b2)


def kernel(x, E, W1, b1, W2, b2):
    batch, k = x.shape
    vocab, emb = E.shape
    hid = W1.shape[1]

    P = _tc_make_table(E.T)

    xt = x.T.astype(jnp.int32)

    W1k = W1.reshape(k, emb, hid)
    W1d = jnp.concatenate([W1k, W1k], axis=1)
    W1d = W1d.reshape(k * 2 * emb, hid).astype(jnp.bfloat16)

    sizes = (6144, 6144, 4096)
    offs = [0]
    for sz in sizes:
        offs.append(offs[-1] + sz)
    outs = []
    for c, bc in enumerate(sizes):
        xt_c = xt[:, offs[c]:offs[c] + bc]
        idx_c = (xt_c & (_QUART - 1)).reshape(1, bc * k)
        g = _sc_gather(P, idx_c)
        g = g.reshape(k, bc, 2 * emb)
        outs.append(_tc_mlp(
            g, xt_c, W1d, b1.reshape(1, -1), W2, b2.reshape(1, -1), 0, bc,
        ))
    return jnp.concatenate(outs, axis=0)


# speedup vs baseline: 1.0604x; 1.0215x over previous
import functools

import jax
import jax.numpy as jnp
from jax.experimental import pallas as pl
from jax.experimental.pallas import tpu as pltpu
from jax.experimental.pallas import tpu_sc as plsc

_QUART = 1 << 18
_COL_BLOCK = 8192
_GATHER_WINDOW = 256
_BATCH_BLOCK = 2048
_CHUNKS = 4


def _pack_bf16(a_ref, b_ref):
    au = jax.lax.bitcast_convert_type(a_ref[...], jnp.uint32)
    bu = jax.lax.bitcast_convert_type(b_ref[...], jnp.uint32)
    return (au & jnp.uint32(0xFFFF0000)) | (bu >> 16)


def _relayout_kernel(p0_ref, p1_ref, p2_ref, p3_ref, p_ref):
    stacked = jnp.concatenate(
        [_pack_bf16(p0_ref, p1_ref), _pack_bf16(p2_ref, p3_ref)], axis=0
    )
    packed = jnp.swapaxes(stacked, 0, 1)
    p_ref[...] = jax.lax.bitcast_convert_type(packed, jnp.int32)


def _tc_make_table(Et):
    emb, vocab = Et.shape
    n_q = _QUART // _COL_BLOCK
    n_all = pl.cdiv(vocab, _COL_BLOCK)

    def part_map(p):
        def index_map(i):
            return (0, jnp.minimum(i + p * n_q, n_all - 1))
        return index_map

    return pl.pallas_call(
        _relayout_kernel,
        grid=(n_q,),
        in_specs=[
            pl.BlockSpec((emb, _COL_BLOCK), part_map(p)) for p in range(4)
        ],
        out_specs=pl.BlockSpec((_COL_BLOCK, 2 * emb), lambda i: (i, 0)),
        out_shape=jax.ShapeDtypeStruct((_QUART, 2 * emb), jnp.int32),
        compiler_params=pltpu.CompilerParams(
            dimension_semantics=("parallel",),
        ),
    )(Et, Et, Et, Et)


def _sc_gather(P, idx):
    n = idx.shape[1]
    width = P.shape[1]
    mesh = plsc.VectorSubcoreMesh(core_axis_name="core", subcore_axis_name="subcore")

    @functools.partial(
        pl.kernel,
        out_type=jax.ShapeDtypeStruct((n, width), P.dtype),
        mesh=mesh,
    )
    def gather_kernel(table_hbm, idx_hbm, out_hbm):
        def body(idx_vmem, out_vmem):
            pltpu.sync_copy(table_hbm.at[idx_vmem.at[0]], out_vmem)

        pltpu.emit_pipeline(
            body,
            grid=(n // _GATHER_WINDOW,),
            in_specs=[
                pl.BlockSpec((1, _GATHER_WINDOW), index_map=lambda i: (0, i))
            ],
            out_specs=[
                pl.BlockSpec((_GATHER_WINDOW, width), index_map=lambda i: (i, 0))
            ],
            core_axis_name=("core", "subcore"),
            dimension_semantics=(pltpu.PARALLEL,),
        )(idx_hbm, out_hbm)

    return gather_kernel(P, idx)


def _mlp_kernel(g_ref, xt_ref, w1_ref, b1_ref, w2_ref, b2_ref, o_ref):
    k_dim, bb, width = g_ref.shape
    emb = width // 2
    lane = jax.lax.broadcasted_iota(jnp.int32, (bb, width), 1)
    parts = []
    for k in range(k_dim):
        gk = jax.lax.bitcast_convert_type(g_ref[k], jnp.uint32)
        part = xt_ref[k][:, None] >> 18
        keep = (lane >= emb) == (part >= 2)
        shifted = (gk << ((part & 1).astype(jnp.uint32) * 16)) \
            & jnp.uint32(0xFFFF0000)
        val = jax.lax.bitcast_convert_type(
            jnp.where(keep, shifted, jnp.uint32(0)), jnp.float32)
        parts.append(val.astype(jnp.bfloat16))
    f = jnp.concatenate(parts, axis=1)
    h = jnp.dot(f, w1_ref[...], preferred_element_type=jnp.float32)
    h = jnp.maximum(h + b1_ref[...], 0.0)
    o = jnp.dot(h, w2_ref[...], preferred_element_type=jnp.float32)
    o_ref[...] = o + b2_ref[...]


def _tc_mlp(g, xt, W1d, b1, W2, b2, start, rows):
    k_dim, b, width = g.shape
    hid = W1d.shape[1]
    out = W2.shape[1]
    s0 = start // _BATCH_BLOCK
    return pl.pallas_call(
        _mlp_kernel,
        grid=(rows // _BATCH_BLOCK,),
        in_specs=[
            pl.BlockSpec((k_dim, _BATCH_BLOCK, width), lambda i: (0, i + s0, 0)),
            pl.BlockSpec((k_dim, _BATCH_BLOCK), lambda i: (0, i + s0)),
            pl.BlockSpec((k_dim * width, hid), lambda i: (0, 0)),
            pl.BlockSpec((1, hid), lambda i: (0, 0)),
            pl.BlockSpec((hid, out), lambda i: (0, 0)),
            pl.BlockSpec((1, out), lambda i: (0, 0)),
        ],
        out_specs=pl.BlockSpec((_BATCH_BLOCK, out), lambda i: (i, 0)),
        out_shape=jax.ShapeDtypeStruct((rows, out), jnp.float32),
        compiler_params=pltpu.CompilerParams(
            dimension_semantics=("parallel",),
        ),
    )(g, xt, W1d, b1, W2, b2)


def kernel(x, E, W1, b1, W2, b2):
    batch, k = x.shape
    vocab, emb = E.shape
    hid = W1.shape[1]

    P = _tc_make_table(E.T)

    xt = x.T.astype(jnp.int32)

    W1k = W1.reshape(k, emb, hid)
    W1d = jnp.concatenate([W1k, W1k], axis=1)
    W1d = W1d.reshape(k * 2 * emb, hid).astype(jnp.bfloat16)

    sizes = (6144, 6144, 4096)
    offs = [0]
    for sz in sizes:
        offs.append(offs[-1] + sz)
    outs = []
    for c, bc in enumerate(sizes):
        xt_c = xt[:, offs[c]:offs[c] + bc]
        idx_c = (xt_c & (_QUART - 1)).reshape(1, bc * k)
        g = _sc_gather(P, idx_c)
        g = g.reshape(k, bc, 2 * emb)
        outs.append(_tc_mlp(
            g, xt_c, W1d, b1.reshape(1, -1), W2, b2.reshape(1, -1), 0, bc,
        ))
    return jnp.concatenate(outs, axis=0)


# speedup vs baseline: 1.0634x; 1.0028x over previous
import functools

import jax
import jax.numpy as jnp
from jax.experimental import pallas as pl
from jax.experimental.pallas import tpu as pltpu
from jax.experimental.pallas import tpu_sc as plsc

_QUART = 1 << 18
_COL_BLOCK = 16384
_GATHER_WINDOW = 256
_BATCH_BLOCK = 2048
_CHUNKS = 4


def _pack_bf16(a_ref, b_ref):
    au = jax.lax.bitcast_convert_type(a_ref[...], jnp.uint32)
    bu = jax.lax.bitcast_convert_type(b_ref[...], jnp.uint32)
    return (au & jnp.uint32(0xFFFF0000)) | (bu >> 16)


def _relayout_kernel(p0_ref, p1_ref, p2_ref, p3_ref, p_ref):
    stacked = jnp.concatenate(
        [_pack_bf16(p0_ref, p1_ref), _pack_bf16(p2_ref, p3_ref)], axis=0
    )
    packed = jnp.swapaxes(stacked, 0, 1)
    p_ref[...] = jax.lax.bitcast_convert_type(packed, jnp.int32)


def _tc_make_table(Et):
    emb, vocab = Et.shape
    n_q = _QUART // _COL_BLOCK
    n_all = pl.cdiv(vocab, _COL_BLOCK)

    def part_map(p):
        def index_map(i):
            return (0, jnp.minimum(i + p * n_q, n_all - 1))
        return index_map

    return pl.pallas_call(
        _relayout_kernel,
        grid=(n_q,),
        in_specs=[
            pl.BlockSpec((emb, _COL_BLOCK), part_map(p)) for p in range(4)
        ],
        out_specs=pl.BlockSpec((_COL_BLOCK, 2 * emb), lambda i: (i, 0)),
        out_shape=jax.ShapeDtypeStruct((_QUART, 2 * emb), jnp.int32),
        compiler_params=pltpu.CompilerParams(
            dimension_semantics=("parallel",),
            vmem_limit_bytes=100 * 1024 * 1024,
        ),
    )(Et, Et, Et, Et)


def _sc_gather(P, idx):
    n = idx.shape[1]
    width = P.shape[1]
    mesh = plsc.VectorSubcoreMesh(core_axis_name="core", subcore_axis_name="subcore")

    @functools.partial(
        pl.kernel,
        out_type=jax.ShapeDtypeStruct((n, width), P.dtype),
        mesh=mesh,
    )
    def gather_kernel(table_hbm, idx_hbm, out_hbm):
        def body(idx_vmem, out_vmem):
            pltpu.sync_copy(table_hbm.at[idx_vmem.at[0]], out_vmem)

        pltpu.emit_pipeline(
            body,
            grid=(n // _GATHER_WINDOW,),
            in_specs=[
                pl.BlockSpec((1, _GATHER_WINDOW), index_map=lambda i: (0, i))
            ],
            out_specs=[
                pl.BlockSpec((_GATHER_WINDOW, width), index_map=lambda i: (i, 0))
            ],
            core_axis_name=("core", "subcore"),
            dimension_semantics=(pltpu.PARALLEL,),
        )(idx_hbm, out_hbm)

    return gather_kernel(P, idx)


def _mlp_kernel(g_ref, xt_ref, w1_ref, b1_ref, w2_ref, b2_ref, o_ref):
    k_dim, bb, width = g_ref.shape
    emb = width // 2
    lane = jax.lax.broadcasted_iota(jnp.int32, (bb, width), 1)
    parts = []
    for k in range(k_dim):
        gk = jax.lax.bitcast_convert_type(g_ref[k], jnp.uint32)
        part = xt_ref[k][:, None] >> 18
        keep = (lane >= emb) == (part >= 2)
        shifted = (gk << ((part & 1).astype(jnp.uint32) * 16)) \
            & jnp.uint32(0xFFFF0000)
        val = jax.lax.bitcast_convert_type(
            jnp.where(keep, shifted, jnp.uint32(0)), jnp.float32)
        parts.append(val.astype(jnp.bfloat16))
    f = jnp.concatenate(parts, axis=1)
    h = jnp.dot(f, w1_ref[...], preferred_element_type=jnp.float32)
    h = jnp.maximum(h + b1_ref[...], 0.0)
    o = jnp.dot(h, w2_ref[...], preferred_element_type=jnp.float32)
    o_ref[...] = o + b2_ref[...]


def _tc_mlp(g, xt, W1d, b1, W2, b2, start, rows):
    k_dim, b, width = g.shape
    hid = W1d.shape[1]
    out = W2.shape[1]
    s0 = start // _BATCH_BLOCK
    return pl.pallas_call(
        _mlp_kernel,
        grid=(rows // _BATCH_BLOCK,),
        in_specs=[
            pl.BlockSpec((k_dim, _BATCH_BLOCK, width), lambda i: (0, i + s0, 0)),
            pl.BlockSpec((k_dim, _BATCH_BLOCK), lambda i: (0, i + s0)),
            pl.BlockSpec((k_dim * width, hid), lambda i: (0, 0)),
            pl.BlockSpec((1, hid), lambda i: (0, 0)),
            pl.BlockSpec((hid, out), lambda i: (0, 0)),
            pl.BlockSpec((1, out), lambda i: (0, 0)),
        ],
        out_specs=pl.BlockSpec((_BATCH_BLOCK, out), lambda i: (i, 0)),
        out_shape=jax.ShapeDtypeStruct((rows, out), jnp.float32),
        compiler_params=pltpu.CompilerParams(
            dimension_semantics=("parallel",),
        ),
    )(g, xt, W1d, b1, W2, b2)


def kernel(x, E, W1, b1, W2, b2):
    batch, k = x.shape
    vocab, emb = E.shape
    hid = W1.shape[1]

    P = _tc_make_table(E.T)

    xt = x.T.astype(jnp.int32)

    W1k = W1.reshape(k, emb, hid)
    W1d = jnp.concatenate([W1k, W1k], axis=1)
    W1d = W1d.reshape(k * 2 * emb, hid).astype(jnp.bfloat16)

    sizes = (6144, 6144, 4096)
    offs = [0]
    for sz in sizes:
        offs.append(offs[-1] + sz)
    outs = []
    for c, bc in enumerate(sizes):
        xt_c = xt[:, offs[c]:offs[c] + bc]
        idx_c = (xt_c & (_QUART - 1)).reshape(1, bc * k)
        g = _sc_gather(P, idx_c)
        g = g.reshape(k, bc, 2 * emb)
        outs.append(_tc_mlp(
            g, xt_c, W1d, b1.reshape(1, -1), W2, b2.reshape(1, -1), 0, bc,
        ))
    return jnp.concatenate(outs, axis=0)
